# baseline (speedup 1.0000x reference)
import functools
import math

import jax
import jax.numpy as jnp
from jax import lax
from jax.experimental import pallas as pl
from jax.experimental.pallas import tpu as pltpu
from jax.experimental.pallas import tpu_sc as plsc

_D = 32
_SCALE = math.sqrt(float(_D))
_NC = 2
_NS = 16
_NW = _NC * _NS
_C = 1024
_NB = 2
_Q = _C // 4


@jax.jit
def _embed(tokens_q, table):
    n = tokens_q.shape[1] * 4
    per_w = n // _NW
    n_chunks = per_w // _C
    n_groups = n_chunks // _NB

    mesh = plsc.VectorSubcoreMesh(core_axis_name="c", subcore_axis_name="s")

    @functools.partial(
        pl.kernel,
        mesh=mesh,
        out_type=jax.ShapeDtypeStruct((n // 4, 128), jnp.float32),
        scratch_types=[
            pltpu.VMEM((_NB, 4, _Q), jnp.int32),
            pltpu.VMEM((_NB, 4, _Q, _D), jnp.float32),
            pltpu.SemaphoreType.DMA((_NB,)),
            pltpu.SemaphoreType.DMA((_NB,)),
        ],
        compiler_params=pltpu.CompilerParams(use_tc_tiling_on_sc=False),
    )
    def emb(tok_hbm, tab_hbm, out_hbm, idx_v, rows_v, gsem, osem):
        wid = lax.axis_index("s") * _NC + lax.axis_index("c")
        base = wid * per_w

        def start_gather(ci, b):
            off4 = pl.multiple_of((base + ci * _C) // 4, _Q)
            for j in range(4):
                pltpu.sync_copy(
                    tok_hbm.at[j, pl.ds(off4, _Q)], idx_v.at[b, j]
                )
            for j in range(4):
                pltpu.make_async_copy(
                    tab_hbm.at[idx_v.at[b, j]], rows_v.at[b, j], gsem.at[b]
                ).start()

        def wait_gather(b):
            for j in range(4):
                pltpu.make_async_copy(
                    tab_hbm.at[idx_v.at[b, j]], rows_v.at[b, j], gsem.at[b]
                ).wait()

        def out_descs(off4, b):
            return [
                pltpu.make_async_copy(
                    rows_v.at[b, j],
                    out_hbm.at[pl.ds(off4, _Q), pl.ds(32 * j, _D)],
                    osem.at[b],
                )
                for j in range(4)
            ]

        for b in range(_NB):
            start_gather(b, b)

        def group(g, carry):
            ci0 = g * _NB
            for b in range(_NB):
                ci = ci0 + b
                off4 = pl.multiple_of((base + ci * _C) // 4, _Q)
                wait_gather(b)

                @plsc.parallel_loop(0, _Q, 1, unroll=2)
                def _scale(i):
                    for j in range(4):
                        rows_v[b, j, i, pl.ds(0, 16)] = (
                            rows_v[b, j, i, pl.ds(0, 16)] * _SCALE
                        )
                        rows_v[b, j, i, pl.ds(16, 16)] = (
                            rows_v[b, j, i, pl.ds(16, 16)] * _SCALE
                        )

                descs = out_descs(off4, b)
                for d in descs:
                    d.start()

                @pl.when(g + 1 < n_groups)
                def _refill():
                    for d in descs:
                        d.wait()
                    start_gather(ci + _NB, b)

            return carry

        lax.fori_loop(0, n_groups, group, 0)

        for b in range(_NB):
            off4 = pl.multiple_of(
                (base + ((n_groups - 1) * _NB + b) * _C) // 4, _Q
            )
            for d in out_descs(off4, b):
                d.wait()

    return emb(tokens_q, table)


def kernel(tokens, table):
    b, s = tokens.shape
    out = _embed(tokens.reshape(-1, 4).T, table)
    return out.reshape(b, s, _D)


# speedup vs baseline: 1.1842x; 1.1842x over previous
import functools
import math

import jax
import jax.numpy as jnp
from jax import lax
from jax.experimental import pallas as pl
from jax.experimental.pallas import tpu as pltpu
from jax.experimental.pallas import tpu_sc as plsc

_D = 32
_SCALE = math.sqrt(float(_D))
_NC = 2
_NS = 16
_NW = _NC * _NS
_C = 640
_NB = 5


@jax.jit
def _embed(tokens_flat, table):
    n = tokens_flat.shape[0]
    per_w = n // _NW
    n_chunks = per_w // _C
    n_groups = n_chunks // _NB

    mesh = plsc.VectorSubcoreMesh(core_axis_name="c", subcore_axis_name="s")

    @functools.partial(
        pl.kernel,
        mesh=mesh,
        out_type=jax.ShapeDtypeStruct((n, _D), jnp.float32),
        scratch_types=[
            pltpu.VMEM((_NB, _C), jnp.int32),
            pltpu.VMEM((_NB, _C, _D), jnp.float32),
            pltpu.SemaphoreType.DMA((_NB,)),
            pltpu.SemaphoreType.DMA((_NB,)),
        ],
        compiler_params=pltpu.CompilerParams(use_tc_tiling_on_sc=False),
    )
    def emb(tok_hbm, tab_hbm, out_hbm, idx_v, rows_v, gsem, osem):
        wid = lax.axis_index("s") * _NC + lax.axis_index("c")
        base = wid * per_w

        def start_gather(ci, b):
            off = base + ci * _C
            pltpu.sync_copy(tok_hbm.at[pl.ds(off, _C)], idx_v.at[b])
            pltpu.make_async_copy(
                tab_hbm.at[idx_v.at[b]], rows_v.at[b], gsem.at[b]
            ).start()

        for b in range(_NB):
            start_gather(b, b)

        def group(g, carry):
            ci0 = g * _NB
            for b in range(_NB):
                ci = ci0 + b
                off = base + ci * _C
                pltpu.make_async_copy(
                    tab_hbm.at[idx_v.at[b]], rows_v.at[b], gsem.at[b]
                ).wait()

                @plsc.parallel_loop(0, _C, 1, unroll=8)
                def _scale(i):
                    rows_v[b, i, pl.ds(0, 16)] = rows_v[b, i, pl.ds(0, 16)] * _SCALE
                    rows_v[b, i, pl.ds(16, 16)] = (
                        rows_v[b, i, pl.ds(16, 16)] * _SCALE
                    )

                out_copy = pltpu.make_async_copy(
                    rows_v.at[b], out_hbm.at[pl.ds(off, _C)], osem.at[b]
                )
                out_copy.start()

                @pl.when(g + 1 < n_groups)
                def _refill():
                    out_copy.wait()
                    start_gather(ci + _NB, b)

            return carry

        lax.fori_loop(0, n_groups, group, 0)

        for b in range(_NB):
            off = base + ((n_groups - 1) * _NB + b) * _C
            pltpu.make_async_copy(
                rows_v.at[b], out_hbm.at[pl.ds(off, _C)], osem.at[b]
            ).wait()

    return emb(tokens_flat, table)


def kernel(tokens, table):
    b, s = tokens.shape
    out = _embed(tokens.reshape(-1), table)
    return out.reshape(b, s, _D)


# speedup vs baseline: 1.2243x; 1.0338x over previous
import functools
import math

import jax
import jax.numpy as jnp
from jax import lax
from jax.experimental import pallas as pl
from jax.experimental.pallas import tpu as pltpu
from jax.experimental.pallas import tpu_sc as plsc

_D = 32
_SCALE = math.sqrt(float(_D))
_NC = 2
_NS = 16
_NW = _NC * _NS
_C = 800
_NB = 4
_LA = 2


@jax.jit
def _embed(tokens_flat, table):
    n = tokens_flat.shape[0]
    per_w = n // _NW
    n_chunks = per_w // _C
    n_groups = n_chunks // _NB

    mesh = plsc.VectorSubcoreMesh(core_axis_name="c", subcore_axis_name="s")

    @functools.partial(
        pl.kernel,
        mesh=mesh,
        out_type=jax.ShapeDtypeStruct((n, _D), jnp.float32),
        scratch_types=[
            pltpu.VMEM((_NB, _C), jnp.int32),
            pltpu.VMEM((_NB, _C, _D), jnp.float32),
            pltpu.SemaphoreType.DMA((_NB,)),
            pltpu.SemaphoreType.DMA((_NB,)),
        ],
        compiler_params=pltpu.CompilerParams(use_tc_tiling_on_sc=False),
    )
    def emb(tok_hbm, tab_hbm, out_hbm, idx_v, rows_v, gsem, osem):
        wid = lax.axis_index("s") * _NC + lax.axis_index("c")
        base = wid * per_w

        def start_gather(ci, b):
            off = base + ci * _C
            pltpu.sync_copy(tok_hbm.at[pl.ds(off, _C)], idx_v.at[b])
            pltpu.make_async_copy(
                tab_hbm.at[idx_v.at[b]], rows_v.at[b], gsem.at[b]
            ).start()

        for b in range(_LA):
            start_gather(b, b)

        def group(g, carry):
            ci0 = g * _NB
            for b in range(_NB):
                ci = ci0 + b
                off = base + ci * _C
                pltpu.make_async_copy(
                    tab_hbm.at[idx_v.at[b]], rows_v.at[b], gsem.at[b]
                ).wait()

                @plsc.parallel_loop(0, _C, 1, unroll=8)
                def _scale(i):
                    rows_v[b, i, pl.ds(0, 16)] = rows_v[b, i, pl.ds(0, 16)] * _SCALE
                    rows_v[b, i, pl.ds(16, 16)] = (
                        rows_v[b, i, pl.ds(16, 16)] * _SCALE
                    )

                pltpu.make_async_copy(
                    rows_v.at[b], out_hbm.at[pl.ds(off, _C)], osem.at[b]
                ).start()

                bb = (b + _LA) % _NB

                @pl.when(ci + _LA < n_chunks)
                def _refill():
                    @pl.when(ci + _LA >= _NB)
                    def _drain_prev():
                        pltpu.make_async_copy(
                            rows_v.at[bb],
                            out_hbm.at[pl.ds(off, _C)],
                            osem.at[bb],
                        ).wait()

                    start_gather(ci + _LA, bb)

            return carry

        lax.fori_loop(0, n_groups, group, 0)

        for b in range(_NB):
            off = base + ((n_groups - 1) * _NB + b) * _C
            pltpu.make_async_copy(
                rows_v.at[b], out_hbm.at[pl.ds(off, _C)], osem.at[b]
            ).wait()

    return emb(tokens_flat, table)


def kernel(tokens, table):
    b, s = tokens.shape
    out = _embed(tokens.reshape(-1), table)
    return out.reshape(b, s, _D)


# speedup vs baseline: 1.2270x; 1.0022x over previous
import functools
import math

import jax
import jax.numpy as jnp
from jax import lax
from jax.experimental import pallas as pl
from jax.experimental.pallas import tpu as pltpu
from jax.experimental.pallas import tpu_sc as plsc

_D = 32
_SCALE = math.sqrt(float(_D))
_NC = 2
_NS = 16
_NW = _NC * _NS
_C = 800
_NB = 4
_LA = 3


@jax.jit
def _embed(tokens_flat, table):
    n = tokens_flat.shape[0]
    per_w = n // _NW
    n_chunks = per_w // _C
    n_groups = n_chunks // _NB

    mesh = plsc.VectorSubcoreMesh(core_axis_name="c", subcore_axis_name="s")

    @functools.partial(
        pl.kernel,
        mesh=mesh,
        out_type=jax.ShapeDtypeStruct((n, _D), jnp.float32),
        scratch_types=[
            pltpu.VMEM((_NB, _C), jnp.int32),
            pltpu.VMEM((_NB, _C, _D), jnp.float32),
            pltpu.SemaphoreType.DMA((_NB,)),
            pltpu.SemaphoreType.DMA((_NB,)),
        ],
        compiler_params=pltpu.CompilerParams(use_tc_tiling_on_sc=False),
    )
    def emb(tok_hbm, tab_hbm, out_hbm, idx_v, rows_v, gsem, osem):
        wid = lax.axis_index("s") * _NC + lax.axis_index("c")
        base = wid * per_w

        def start_gather(ci, b):
            off = base + ci * _C
            pltpu.sync_copy(tok_hbm.at[pl.ds(off, _C)], idx_v.at[b])
            pltpu.make_async_copy(
                tab_hbm.at[idx_v.at[b]], rows_v.at[b], gsem.at[b]
            ).start()

        for b in range(_LA):
            start_gather(b, b)

        def group(g, carry):
            ci0 = g * _NB
            for b in range(_NB):
                ci = ci0 + b
                off = base + ci * _C
                pltpu.make_async_copy(
                    tab_hbm.at[idx_v.at[b]], rows_v.at[b], gsem.at[b]
                ).wait()

                @plsc.parallel_loop(0, _C, 1, unroll=8)
                def _scale(i):
                    rows_v[b, i, pl.ds(0, 16)] = rows_v[b, i, pl.ds(0, 16)] * _SCALE
                    rows_v[b, i, pl.ds(16, 16)] = (
                        rows_v[b, i, pl.ds(16, 16)] * _SCALE
                    )

                pltpu.make_async_copy(
                    rows_v.at[b], out_hbm.at[pl.ds(off, _C)], osem.at[b]
                ).start()

                bb = (b + _LA) % _NB

                @pl.when(ci + _LA < n_chunks)
                def _refill():
                    @pl.when(ci + _LA >= _NB)
                    def _drain_prev():
                        pltpu.make_async_copy(
                            rows_v.at[bb],
                            out_hbm.at[pl.ds(off, _C)],
                            osem.at[bb],
                        ).wait()

                    start_gather(ci + _LA, bb)

            return carry

        lax.fori_loop(0, n_groups, group, 0)

        for b in range(_NB):
            off = base + ((n_groups - 1) * _NB + b) * _C
            pltpu.make_async_copy(
                rows_v.at[b], out_hbm.at[pl.ds(off, _C)], osem.at[b]
            ).wait()

    return emb(tokens_flat, table)


def kernel(tokens, table):
    b, s = tokens.shape
    out = _embed(tokens.reshape(-1), table)
    return out.reshape(b, s, _D)
